# Initial kernel scaffold; baseline (speedup 1.0000x reference)
import jax
import jax.numpy as jnp
from jax.experimental import pallas as pl


def kernel(x, edge_index, W1_mean, b1_mean, W1_var, b1_var, W_mean_out, b_mean_out, W_var_out, b_var_out):
    raise NotImplementedError("write your pallas kernel here")


# speedup vs baseline: 10.7794x; 10.7794x over previous
import functools

import jax
import jax.numpy as jnp
from jax import lax
from jax.experimental import pallas as pl
from jax.experimental.pallas import tpu as pltpu
from jax.experimental.pallas import tpu_sc as plsc

_N = 10000
_E = 320000
_D = 128
_DA = 144

_NC = 2
_NS = 16
_NW = _NC * _NS
_BATCH = 100
_BPW = (_E // _BATCH) // _NW
_GRP = 10
_RPT = _N // _NS

_mesh = plsc.VectorSubcoreMesh(
    core_axis_name="c", subcore_axis_name="s", num_cores=_NC, num_subcores=_NS
)


@functools.partial(
    pl.kernel,
    out_type=jax.ShapeDtypeStruct((_NC, _N, _DA), jnp.float32),
    mesh=_mesh,
    compiler_params=pltpu.CompilerParams(use_tc_tiling_on_sc=False),
    scratch_types=[
        pltpu.VMEM_SHARED((_N, _DA), jnp.float32),
        pltpu.VMEM((_BPW, _BATCH), jnp.int32),
        pltpu.VMEM((_GRP, _BATCH), jnp.int32),
        pltpu.VMEM((2, _BATCH, _DA), jnp.float32),
        pltpu.SemaphoreType.DMA,
        pltpu.SemaphoreType.DMA,
    ],
)
def _sc_aggregate(src3d, dst3d, xaug, zeros_hbm, out_acc,
                  acc_sh, src_v, dstg_v, rows_v, sem0, sem1):
    c = lax.axis_index("c")
    s = lax.axis_index("s")
    wid = s * _NC + c
    sems = (sem0, sem1)

    r0 = s * _RPT
    pltpu.sync_copy(zeros_hbm, rows_v.at[0])
    for j in range(_RPT // _BATCH):
        pltpu.sync_copy(rows_v.at[0], acc_sh.at[pl.ds(r0 + j * _BATCH, _BATCH)])
    rem = _RPT % _BATCH
    if rem:
        pltpu.sync_copy(rows_v.at[0, pl.ds(0, rem)],
                        acc_sh.at[pl.ds(r0 + _RPT - rem, rem)])

    pltpu.sync_copy(src3d.at[wid], src_v)
    plsc.subcore_barrier()

    def gather(k, slot):
        return pltpu.async_copy(xaug.at[src_v.at[k]], rows_v.at[slot],
                                sems[slot])

    def gather_wait(k, slot):
        pltpu.make_async_copy(xaug.at[src_v.at[k]], rows_v.at[slot],
                              sems[slot]).wait()

    gather(0, 0)
    gather(1, 1)

    def group(g, last):
        pltpu.sync_copy(dst3d.at[wid, pl.ds(g * _GRP, _GRP)], dstg_v)
        for j in range(_GRP):
            k = g * _GRP + j
            slot = j % 2
            gather_wait(k, slot)
            pltpu.sync_copy(rows_v.at[slot], acc_sh.at[dstg_v.at[j]],
                            add=True)
            if not (last and j >= _GRP - 2):
                gather(k + 2, slot)

    lax.fori_loop(0, _BPW // _GRP - 1, lambda g, _: (group(g, False), 0)[1], 0)
    group(_BPW // _GRP - 1, True)
    plsc.subcore_barrier()

    pltpu.sync_copy(acc_sh.at[pl.ds(r0, _RPT)], out_acc.at[c, pl.ds(r0, _RPT)])


_R = 1000


def _tc_heads_body(acc_ref, w1m, b1m, w1v, b1v, wmo, bmo, wvo, bvo,
                   mean_ref, var_ref):
    sums = acc_ref[0, :, :_D] + acc_ref[1, :, :_D]
    cnt = acc_ref[0, :, _D:_D + 1] + acc_ref[1, :, _D:_D + 1]
    agg = sums / jnp.maximum(cnt, 1.0)
    hm = jnp.maximum(
        jnp.dot(agg, w1m[...], preferred_element_type=jnp.float32) + b1m[...],
        0.0)
    mean_ref[...] = (
        jnp.dot(hm, wmo[...], preferred_element_type=jnp.float32) + bmo[...])
    hv = jnp.maximum(
        jnp.dot(agg, w1v[...], preferred_element_type=jnp.float32) + b1v[...],
        0.0)
    var_ref[...] = (
        jnp.dot(hv, wvo[...], preferred_element_type=jnp.float32) + bvo[...])


def _tc_heads(acc, W1m, b1m, W1v, b1v, Wmo, bmo, Wvo, bvo):
    wspec = pl.BlockSpec((_D, _D), lambda i: (0, 0))
    bspec = pl.BlockSpec((1, _D), lambda i: (0, 0))
    return pl.pallas_call(
        _tc_heads_body,
        grid=(_N // _R,),
        in_specs=[
            pl.BlockSpec((_NC, _R, _DA), lambda i: (0, i, 0)),
            wspec, bspec, wspec, bspec, wspec, bspec, wspec, bspec,
        ],
        out_specs=[
            pl.BlockSpec((_R, _D), lambda i: (i, 0)),
            pl.BlockSpec((_R, _D), lambda i: (i, 0)),
        ],
        out_shape=[
            jax.ShapeDtypeStruct((_N, _D), jnp.float32),
            jax.ShapeDtypeStruct((_N, _D), jnp.float32),
        ],
    )(acc, W1m, b1m, W1v, b1v, Wmo, bmo, Wvo, bvo)


@jax.jit
def kernel(x, edge_index, W1_mean, b1_mean, W1_var, b1_var,
           W_mean_out, b_mean_out, W_var_out, b_var_out):
    src3d = edge_index[0].reshape(_NW, _BPW, _BATCH)
    dst3d = edge_index[1].reshape(_NW, _BPW, _BATCH)
    xaug = jnp.concatenate([x, jnp.ones((_N, _DA - _D), jnp.float32)], axis=1)
    zeros = jnp.zeros((_BATCH, _DA), jnp.float32)
    acc = _sc_aggregate(src3d, dst3d, xaug, zeros)
    mean, variance = _tc_heads(
        acc, W1_mean, b1_mean.reshape(1, _D), W1_var, b1_var.reshape(1, _D),
        W_mean_out, b_mean_out.reshape(1, _D), W_var_out,
        b_var_out.reshape(1, _D))
    return (mean, variance)
